# Initial kernel scaffold; baseline (speedup 1.0000x reference)
import jax
import jax.numpy as jnp
from jax.experimental import pallas as pl


def kernel(x, edge_index, params):
    raise NotImplementedError("write your pallas kernel here")


# speedup vs baseline: 4.5422x; 4.5422x over previous
import functools

import jax
import jax.numpy as jnp
from jax import lax
from jax.experimental import pallas as pl
from jax.experimental.pallas import tpu as pltpu
from jax.experimental.pallas import tpu_sc as plsc

_NC = 2
_NS = 16
_BLK = 1000


def _dot_t(a, b):
    return lax.dot_general(a, b, (((1,), (1,)), ((), ())),
                           preferred_element_type=jnp.float32)


def _mlp_tc(x, w1, b1, w2, b2):
    n, din = x.shape
    hid = w1.shape[0]
    dout = w2.shape[0]

    def body(x_ref, w1_ref, b1_ref, w2_ref, b2_ref, o_ref):
        h = jnp.maximum(_dot_t(x_ref[...], w1_ref[...]) + b1_ref[...], 0.0)
        o_ref[...] = jnp.maximum(_dot_t(h, w2_ref[...]) + b2_ref[...], 0.0)

    return pl.pallas_call(
        body,
        grid=(n // _BLK,),
        in_specs=[
            pl.BlockSpec((_BLK, din), lambda i: (i, 0)),
            pl.BlockSpec((hid, din), lambda i: (0, 0)),
            pl.BlockSpec((1, hid), lambda i: (0, 0)),
            pl.BlockSpec((dout, hid), lambda i: (0, 0)),
            pl.BlockSpec((1, dout), lambda i: (0, 0)),
        ],
        out_specs=pl.BlockSpec((_BLK, dout), lambda i: (i, 0)),
        out_shape=jax.ShapeDtypeStruct((n, dout), jnp.float32),
    )(x, w1, b1, w2, b2)


def _update_tc(p0, p1, x, w1a, w1b, b1, w2, b2):
    n, d = x.shape
    hid = w1a.shape[0]
    dout = w2.shape[0]

    def body(p0_ref, p1_ref, x_ref, w1a_ref, w1b_ref, b1_ref, w2_ref,
             b2_ref, o_ref):
        aggr = p0_ref[...] + p1_ref[...]
        h = (_dot_t(aggr, w1a_ref[...]) + _dot_t(x_ref[...], w1b_ref[...])
             + b1_ref[...])
        h = jnp.maximum(h, 0.0)
        o_ref[...] = jnp.maximum(_dot_t(h, w2_ref[...]) + b2_ref[...], 0.0)

    return pl.pallas_call(
        body,
        grid=(n // _BLK,),
        in_specs=[
            pl.BlockSpec((_BLK, d), lambda i: (i, 0)),
            pl.BlockSpec((_BLK, d), lambda i: (i, 0)),
            pl.BlockSpec((_BLK, d), lambda i: (i, 0)),
            pl.BlockSpec((hid, d), lambda i: (0, 0)),
            pl.BlockSpec((hid, d), lambda i: (0, 0)),
            pl.BlockSpec((1, hid), lambda i: (0, 0)),
            pl.BlockSpec((dout, hid), lambda i: (0, 0)),
            pl.BlockSpec((1, dout), lambda i: (0, 0)),
        ],
        out_specs=pl.BlockSpec((_BLK, dout), lambda i: (i, 0)),
        out_shape=jax.ShapeDtypeStruct((n, dout), jnp.float32),
    )(p0, p1, x, w1a, w1b, b1, w2, b2)


def _edge_aggregate(msg, src, dst, zeros, n_pad):
    n, d = msg.shape
    e = src.shape[0]
    nw = _NC * _NS
    per_w = e // nw
    chunk = 80
    n_chunks = per_w // chunk
    rows_per_s = n_pad // _NS

    mesh = plsc.VectorSubcoreMesh(core_axis_name="c", subcore_axis_name="s",
                                  num_cores=_NC, num_subcores=_NS)

    @functools.partial(
        pl.kernel,
        mesh=mesh,
        out_type=[jax.ShapeDtypeStruct((n_pad, d), jnp.float32),
                  jax.ShapeDtypeStruct((n_pad, d), jnp.float32)],
        scratch_types=[
            pltpu.VMEM((chunk,), jnp.int32),
            pltpu.VMEM((chunk,), jnp.int32),
            pltpu.VMEM((chunk, d), jnp.float32),
            pltpu.VMEM_SHARED((n_pad, d), jnp.float32),
            pltpu.SemaphoreType.DMA,
        ],
    )
    def body(msg_hbm, src_hbm, dst_hbm, zero_hbm, out0_hbm, out1_hbm,
             sidx, didx, rows, acc, sem):
        c = lax.axis_index("c")
        s = lax.axis_index("s")
        wid = s * _NC + c
        r0 = s * rows_per_s
        pltpu.sync_copy(zero_hbm.at[pl.ds(r0, rows_per_s)],
                        acc.at[pl.ds(r0, rows_per_s)])
        plsc.subcore_barrier()
        ebase = wid * per_w

        def step(i, carry):
            base = ebase + i * chunk
            pltpu.sync_copy(src_hbm.at[pl.ds(base, chunk)], sidx)
            pltpu.sync_copy(dst_hbm.at[pl.ds(base, chunk)], didx)
            pltpu.async_copy(msg_hbm.at[sidx], rows, sem).wait()
            pltpu.sync_copy(rows, acc.at[didx], add=True)
            return carry

        lax.fori_loop(0, n_chunks, step, 0)
        plsc.subcore_barrier()

        @pl.when(c == 0)
        def _():
            pltpu.sync_copy(acc.at[pl.ds(r0, rows_per_s)],
                            out0_hbm.at[pl.ds(r0, rows_per_s)])

        @pl.when(c == 1)
        def _():
            pltpu.sync_copy(acc.at[pl.ds(r0, rows_per_s)],
                            out1_hbm.at[pl.ds(r0, rows_per_s)])

    return body(msg, src, dst, zeros)


def kernel(x, edge_index, params):
    src = edge_index[0].astype(jnp.int32)
    dst = edge_index[1].astype(jnp.int32)
    n, d = x.shape
    n_pad = ((n + 8 * _NS - 1) // (8 * _NS)) * (8 * _NS)
    zeros = jnp.zeros((n_pad, d), jnp.float32)
    for p in params:
        m, u = p['mlp'], p['update']
        msg = _mlp_tc(x, m['W1'], m['b1'].reshape(1, -1),
                      m['W2'], m['b2'].reshape(1, -1))
        p0, p1 = _edge_aggregate(msg, src, dst, zeros, n_pad)
        x = _update_tc(p0, p1, x,
                       u['W1'][:, :d], u['W1'][:, d:],
                       u['b1'].reshape(1, -1), u['W2'],
                       u['b2'].reshape(1, -1))
    return x


# speedup vs baseline: 10.2031x; 2.2463x over previous
import functools

import jax
import jax.numpy as jnp
from jax import lax
from jax.experimental import pallas as pl
from jax.experimental.pallas import tpu as pltpu
from jax.experimental.pallas import tpu_sc as plsc

_NC = 2
_NS = 16
_BLK = 1000


def _dot_t(a, b):
    return lax.dot_general(a, b, (((1,), (1,)), ((), ())),
                           preferred_element_type=jnp.float32)


def _mlp_tc(x, w1, b1, w2, b2):
    n, din = x.shape
    hid = w1.shape[0]
    dout = w2.shape[0]

    def body(x_ref, w1_ref, b1_ref, w2_ref, b2_ref, o_ref):
        h = jnp.maximum(_dot_t(x_ref[...], w1_ref[...]) + b1_ref[...], 0.0)
        o_ref[...] = jnp.maximum(_dot_t(h, w2_ref[...]) + b2_ref[...], 0.0)

    return pl.pallas_call(
        body,
        grid=(n // _BLK,),
        in_specs=[
            pl.BlockSpec((_BLK, din), lambda i: (i, 0)),
            pl.BlockSpec((hid, din), lambda i: (0, 0)),
            pl.BlockSpec((1, hid), lambda i: (0, 0)),
            pl.BlockSpec((dout, hid), lambda i: (0, 0)),
            pl.BlockSpec((1, dout), lambda i: (0, 0)),
        ],
        out_specs=pl.BlockSpec((_BLK, dout), lambda i: (i, 0)),
        out_shape=jax.ShapeDtypeStruct((n, dout), jnp.float32),
    )(x, w1, b1, w2, b2)


def _update_tc(p0, p1, x, w1a, w1b, b1, w2, b2):
    n, d = x.shape
    hid = w1a.shape[0]
    dout = w2.shape[0]

    def body(p0_ref, p1_ref, x_ref, w1a_ref, w1b_ref, b1_ref, w2_ref,
             b2_ref, o_ref):
        aggr = p0_ref[...] + p1_ref[...]
        h = (_dot_t(aggr, w1a_ref[...]) + _dot_t(x_ref[...], w1b_ref[...])
             + b1_ref[...])
        h = jnp.maximum(h, 0.0)
        o_ref[...] = jnp.maximum(_dot_t(h, w2_ref[...]) + b2_ref[...], 0.0)

    return pl.pallas_call(
        body,
        grid=(n // _BLK,),
        in_specs=[
            pl.BlockSpec((_BLK, d), lambda i: (i, 0)),
            pl.BlockSpec((_BLK, d), lambda i: (i, 0)),
            pl.BlockSpec((_BLK, d), lambda i: (i, 0)),
            pl.BlockSpec((hid, d), lambda i: (0, 0)),
            pl.BlockSpec((hid, d), lambda i: (0, 0)),
            pl.BlockSpec((1, hid), lambda i: (0, 0)),
            pl.BlockSpec((dout, hid), lambda i: (0, 0)),
            pl.BlockSpec((1, dout), lambda i: (0, 0)),
        ],
        out_specs=pl.BlockSpec((_BLK, dout), lambda i: (i, 0)),
        out_shape=jax.ShapeDtypeStruct((n, dout), jnp.float32),
    )(p0, p1, x, w1a, w1b, b1, w2, b2)


def _edge_aggregate(msg, src, dst, zeros, n_pad):
    n, d = msg.shape
    nw, n_chunks, chunk = dst.shape
    per_w = n_chunks * chunk
    rows_per_s = n_pad // _NS

    mesh = plsc.VectorSubcoreMesh(core_axis_name="c", subcore_axis_name="s",
                                  num_cores=_NC, num_subcores=_NS)

    @functools.partial(
        pl.kernel,
        mesh=mesh,
        out_type=[jax.ShapeDtypeStruct((n_pad, d), jnp.float32),
                  jax.ShapeDtypeStruct((n_pad, d), jnp.float32)],
        scratch_types=[
            pltpu.VMEM((n_chunks * chunk,), jnp.int32),
            pltpu.VMEM((n_chunks, chunk), jnp.int32),
            pltpu.VMEM((chunk, d), jnp.float32),
            pltpu.VMEM((chunk, d), jnp.float32),
            pltpu.VMEM_SHARED((n_pad, d), jnp.float32),
            pltpu.SemaphoreType.DMA,
            pltpu.SemaphoreType.DMA,
            pltpu.SemaphoreType.DMA,
        ],
    )
    def body(msg_hbm, src_flat_hbm, dst_hbm, zero_hbm, out0_hbm, out1_hbm,
             sidx, didx, rows0, rows1, acc, isem, gsem0, gsem1):
        c = lax.axis_index("c")
        s = lax.axis_index("s")
        wid = s * _NC + c
        r0 = s * rows_per_s
        rows = (rows0, rows1)
        gsem = (gsem0, gsem1)
        icp1 = pltpu.async_copy(src_flat_hbm.at[pl.ds(wid * per_w, per_w)],
                                sidx, isem)
        icp2 = pltpu.async_copy(dst_hbm.at[wid], didx, isem)
        pltpu.sync_copy(zero_hbm.at[pl.ds(r0, rows_per_s)],
                        acc.at[pl.ds(r0, rows_per_s)])
        icp1.wait()
        icp2.wait()
        plsc.subcore_barrier()

        def gather(g, b):
            return pltpu.async_copy(msg_hbm.at[sidx.at[pl.ds(g * chunk, chunk)]],
                                    rows[b], gsem[b])

        def gather_wait(g, b):
            pltpu.make_async_copy(msg_hbm.at[sidx.at[pl.ds(g * chunk, chunk)]],
                                  rows[b], gsem[b]).wait()

        def scatter(g, b):
            pltpu.sync_copy(rows[b], acc.at[didx.at[g]], add=True)

        gather(0, 0)

        def pair(j, carry):
            for k in (0, 1):
                g = 2 * j + k
                gather(g + 1, 1 - k)
                gather_wait(g, k)
                scatter(g, k)
            return carry

        lax.fori_loop(0, (n_chunks - 1) // 2, pair, 0)
        gather_wait(n_chunks - 1, 0)
        scatter(n_chunks - 1, 0)
        plsc.subcore_barrier()

        @pl.when(c == 0)
        def _():
            pltpu.sync_copy(acc.at[pl.ds(r0, rows_per_s)],
                            out0_hbm.at[pl.ds(r0, rows_per_s)])

        @pl.when(c == 1)
        def _():
            pltpu.sync_copy(acc.at[pl.ds(r0, rows_per_s)],
                            out1_hbm.at[pl.ds(r0, rows_per_s)])

    return body(msg, src, dst, zeros)


def kernel(x, edge_index, params):
    n, d = x.shape
    nw = _NC * _NS
    e = edge_index.shape[1]
    chunk = 80
    src = edge_index[0].astype(jnp.int32)
    dst = edge_index[1].astype(jnp.int32).reshape(nw, e // (nw * chunk), chunk)
    n_pad = ((n + 8 * _NS - 1) // (8 * _NS)) * (8 * _NS)
    zeros = jnp.zeros((n_pad, d), jnp.float32)
    for p in params:
        m, u = p['mlp'], p['update']
        msg = _mlp_tc(x, m['W1'], m['b1'].reshape(1, -1),
                      m['W2'], m['b2'].reshape(1, -1))
        p0, p1 = _edge_aggregate(msg, src, dst, zeros, n_pad)
        x = _update_tc(p0, p1, x,
                       u['W1'][:, :d], u['W1'][:, d:],
                       u['b1'].reshape(1, -1), u['W2'],
                       u['b2'].reshape(1, -1))
    return x


# speedup vs baseline: 11.5669x; 1.1337x over previous
import functools

import jax
import jax.numpy as jnp
from jax import lax
from jax.experimental import pallas as pl
from jax.experimental.pallas import tpu as pltpu
from jax.experimental.pallas import tpu_sc as plsc

_NC = 2
_NS = 16
_BLK = 1000


def _dot_t(a, b):
    return lax.dot_general(a, b, (((1,), (1,)), ((), ())),
                           preferred_element_type=jnp.float32)


def _mlp_tc(x, w1, b1, w2, b2):
    n, din = x.shape
    hid = w1.shape[0]
    dout = w2.shape[0]

    def body(x_ref, w1_ref, b1_ref, w2_ref, b2_ref, o_ref):
        h = jnp.maximum(_dot_t(x_ref[...], w1_ref[...]) + b1_ref[...], 0.0)
        o_ref[...] = jnp.maximum(_dot_t(h, w2_ref[...]) + b2_ref[...], 0.0)

    return pl.pallas_call(
        body,
        grid=(n // _BLK,),
        in_specs=[
            pl.BlockSpec((_BLK, din), lambda i: (i, 0)),
            pl.BlockSpec((hid, din), lambda i: (0, 0)),
            pl.BlockSpec((1, hid), lambda i: (0, 0)),
            pl.BlockSpec((dout, hid), lambda i: (0, 0)),
            pl.BlockSpec((1, dout), lambda i: (0, 0)),
        ],
        out_specs=pl.BlockSpec((_BLK, dout), lambda i: (i, 0)),
        out_shape=jax.ShapeDtypeStruct((n, dout), jnp.float32),
    )(x, w1, b1, w2, b2)


def _update_tc(p0, p1, x, w1a, w1b, b1, w2, b2):
    n, d = x.shape
    hid = w1a.shape[0]
    dout = w2.shape[0]

    def body(p0_ref, p1_ref, x_ref, w1a_ref, w1b_ref, b1_ref, w2_ref,
             b2_ref, o_ref):
        aggr = p0_ref[...] + p1_ref[...]
        h = (_dot_t(aggr, w1a_ref[...]) + _dot_t(x_ref[...], w1b_ref[...])
             + b1_ref[...])
        h = jnp.maximum(h, 0.0)
        o_ref[...] = jnp.maximum(_dot_t(h, w2_ref[...]) + b2_ref[...], 0.0)

    return pl.pallas_call(
        body,
        grid=(n // _BLK,),
        in_specs=[
            pl.BlockSpec((_BLK, d), lambda i: (i, 0)),
            pl.BlockSpec((_BLK, d), lambda i: (i, 0)),
            pl.BlockSpec((_BLK, d), lambda i: (i, 0)),
            pl.BlockSpec((hid, d), lambda i: (0, 0)),
            pl.BlockSpec((hid, d), lambda i: (0, 0)),
            pl.BlockSpec((1, hid), lambda i: (0, 0)),
            pl.BlockSpec((dout, hid), lambda i: (0, 0)),
            pl.BlockSpec((1, dout), lambda i: (0, 0)),
        ],
        out_specs=pl.BlockSpec((_BLK, dout), lambda i: (i, 0)),
        out_shape=jax.ShapeDtypeStruct((n, dout), jnp.float32),
    )(p0, p1, x, w1a, w1b, b1, w2, b2)


def _edge_aggregate(msg, src, dst, zeros, n_pad):
    n, d = msg.shape
    nw, n_chunks, chunk = dst.shape
    per_w = n_chunks * chunk
    rows_per_s = n_pad // _NS

    mesh = plsc.VectorSubcoreMesh(core_axis_name="c", subcore_axis_name="s",
                                  num_cores=_NC, num_subcores=_NS)

    @functools.partial(
        pl.kernel,
        mesh=mesh,
        out_type=[jax.ShapeDtypeStruct((n_pad, d), jnp.float32),
                  jax.ShapeDtypeStruct((n_pad, d), jnp.float32)],
        scratch_types=[
            pltpu.VMEM((n_chunks * chunk,), jnp.int32),
            pltpu.VMEM((3, chunk), jnp.int32),
            pltpu.VMEM((chunk, d), jnp.float32),
            pltpu.VMEM((chunk, d), jnp.float32),
            pltpu.VMEM((chunk, d), jnp.float32),
            pltpu.VMEM_SHARED((n_pad, d), jnp.float32),
            pltpu.SemaphoreType.DMA,
            (pltpu.SemaphoreType.DMA,) * 3,
            (pltpu.SemaphoreType.DMA,) * 3,
            (pltpu.SemaphoreType.DMA,) * 3,
        ],
    )
    def body(msg_hbm, src_flat_hbm, dst_hbm, zero_hbm, out0_hbm, out1_hbm,
             sidx, didx, rows0, rows1, rows2, acc, isem, idsem, gsem, ssem):
        c = lax.axis_index("c")
        s = lax.axis_index("s")
        wid = s * _NC + c
        r0 = s * rows_per_s
        rows = (rows0, rows1, rows2)
        icp = pltpu.async_copy(src_flat_hbm.at[pl.ds(wid * per_w, per_w)],
                               sidx, isem)
        pltpu.sync_copy(zero_hbm.at[pl.ds(r0, rows_per_s)],
                        acc.at[pl.ds(r0, rows_per_s)])
        icp.wait()
        plsc.subcore_barrier()

        def didx_load(g, b):
            pltpu.async_copy(dst_hbm.at[wid, g], didx.at[b], idsem[b])

        def didx_wait(g, b):
            pltpu.make_async_copy(dst_hbm.at[wid, g], didx.at[b],
                                  idsem[b]).wait()

        def gather(g, b):
            pltpu.async_copy(msg_hbm.at[sidx.at[pl.ds(g * chunk, chunk)]],
                             rows[b], gsem[b])

        def gather_wait(g, b):
            pltpu.make_async_copy(msg_hbm.at[sidx.at[pl.ds(g * chunk, chunk)]],
                                  rows[b], gsem[b]).wait()

        def scatter(g, b):
            pltpu.async_copy(rows[b], acc.at[didx.at[b]], ssem[b], add=True)

        def scatter_wait(b):
            pltpu.make_async_copy(rows[b], acc.at[didx.at[b]], ssem[b]).wait()

        def stage(g, b, bn):
            gather_wait(g, b)
            didx_wait(g, b)
            scatter(g, b)

            @pl.when(g + 2 < n_chunks)
            def _():
                scatter_wait(bn)
                didx_load(g + 2, bn)
                gather(g + 2, bn)

        didx_load(0, 0)
        didx_load(1, 1)
        gather(0, 0)
        gather(1, 1)
        gather_wait(0, 0)
        didx_wait(0, 0)
        scatter(0, 0)
        didx_load(2, 2)
        gather(2, 2)
        stage(1, 1, 0)

        def triple(j, carry):
            for k in (0, 1, 2):
                g = 2 + 3 * j + k
                stage(g, (2 + k) % 3, (4 + k) % 3)
            return carry

        lax.fori_loop(0, (n_chunks - 2) // 3, triple, 0)
        for g in range(n_chunks - (n_chunks - 2) % 3, n_chunks):
            stage(g, g % 3, (g + 2) % 3)
        scatter_wait((n_chunks - 3) % 3)
        scatter_wait((n_chunks - 2) % 3)
        scatter_wait((n_chunks - 1) % 3)
        plsc.subcore_barrier()

        @pl.when(c == 0)
        def _():
            pltpu.sync_copy(acc.at[pl.ds(r0, rows_per_s)],
                            out0_hbm.at[pl.ds(r0, rows_per_s)])

        @pl.when(c == 1)
        def _():
            pltpu.sync_copy(acc.at[pl.ds(r0, rows_per_s)],
                            out1_hbm.at[pl.ds(r0, rows_per_s)])

    return body(msg, src, dst, zeros)


def kernel(x, edge_index, params):
    n, d = x.shape
    nw = _NC * _NS
    e = edge_index.shape[1]
    chunk = 80
    src = edge_index[0].astype(jnp.int32)
    dst = edge_index[1].astype(jnp.int32).reshape(nw, e // (nw * chunk), chunk)
    n_pad = ((n + 8 * _NS - 1) // (8 * _NS)) * (8 * _NS)
    zeros = jnp.zeros((n_pad, d), jnp.float32)
    for p in params:
        m, u = p['mlp'], p['update']
        msg = _mlp_tc(x, m['W1'], m['b1'].reshape(1, -1),
                      m['W2'], m['b2'].reshape(1, -1))
        p0, p1 = _edge_aggregate(msg, src, dst, zeros, n_pad)
        x = _update_tc(p0, p1, x,
                       u['W1'][:, :d], u['W1'][:, d:],
                       u['b1'].reshape(1, -1), u['W2'],
                       u['b2'].reshape(1, -1))
    return x
